# Initial kernel scaffold; baseline (speedup 1.0000x reference)
import jax
import jax.numpy as jnp
from jax.experimental import pallas as pl


def kernel(x, edge_index, W, b, Wf, bf):
    raise NotImplementedError("write your pallas kernel here")


# speedup vs baseline: 32.2994x; 32.2994x over previous
import functools

import jax
import jax.numpy as jnp
from jax import lax
from jax.experimental import pallas as pl
from jax.experimental.pallas import tpu as pltpu
from jax.experimental.pallas import tpu_sc as plsc

N_NODES = 10000
N_EDGES = 320000
D_IN = 128
D_HID = 64

NC, NS = 2, 16
NW = NC * NS
EPT = N_EDGES // NW
BLK = 128
NBLK = -(-EPT // BLK)
PAD = NBLK * BLK - EPT
R_SH = 10240
RPT = R_SH // NS

def _deg_body(dstb_hbm, zeros_hbm, out_hbm, idx_v, ones_v, deg_sh):
    c = lax.axis_index("c")
    s = lax.axis_index("s")
    wid = s * NC + c
    pltpu.sync_copy(zeros_hbm.at[pl.ds(s * RPT, RPT)], deg_sh.at[pl.ds(s * RPT, RPT)])
    pltpu.sync_copy(dstb_hbm.at[wid], idx_v)
    for i in range(BLK // 16):
        ones_v[pl.ds(i * 16, 16)] = jnp.ones((16,), jnp.float32)
    plsc.subcore_barrier()

    @pl.loop(0, NBLK)
    def _count(j):
        pltpu.sync_copy(ones_v, deg_sh.at[idx_v.at[j]], add=True)

    plsc.subcore_barrier()
    pltpu.sync_copy(deg_sh.at[pl.ds(s * RPT, RPT)],
                    out_hbm.at[c, pl.ds(s * RPT, RPT)])


def _msg_body(srcb_hbm, dstb_hbm, s_hbm, zeros_hbm, out_hbm,
              si_v, di_v, rows0, rows1, acc_sh, sem0, sem1):
    c = lax.axis_index("c")
    s = lax.axis_index("s")
    wid = s * NC + c
    pltpu.sync_copy(zeros_hbm.at[pl.ds(s * RPT, RPT)], acc_sh.at[pl.ds(s * RPT, RPT)])
    pltpu.sync_copy(srcb_hbm.at[wid], si_v)
    pltpu.sync_copy(dstb_hbm.at[wid], di_v)
    plsc.subcore_barrier()

    @pl.loop(0, NBLK - 1, step=2)
    def _pairs(j):
        cp0 = pltpu.async_copy(s_hbm.at[si_v.at[j]], rows0, sem0)
        cp1 = pltpu.async_copy(s_hbm.at[si_v.at[j + 1]], rows1, sem1)
        cp0.wait()
        pltpu.sync_copy(rows0, acc_sh.at[di_v.at[j]], add=True)
        cp1.wait()
        pltpu.sync_copy(rows1, acc_sh.at[di_v.at[j + 1]], add=True)

    cpt = pltpu.async_copy(s_hbm.at[si_v.at[NBLK - 1]], rows0, sem0)
    cpt.wait()
    pltpu.sync_copy(rows0, acc_sh.at[di_v.at[NBLK - 1]], add=True)
    plsc.subcore_barrier()
    pltpu.sync_copy(acc_sh.at[pl.ds(s * RPT, RPT)],
                    out_hbm.at[c, pl.ds(s * RPT, RPT)])


_ROWS_BLK = 1000
_GRID = N_NODES // _ROWS_BLK


def _feat_body(x_ref, w_ref, deg_ref, s_ref, g_ref):
    g = lax.rsqrt(deg_ref[...])
    h = jnp.dot(x_ref[...], w_ref[...], preferred_element_type=jnp.float32)
    s_ref[...] = h * g
    g_ref[...] = g


_feat_call = pl.pallas_call(
    _feat_body,
    grid=(_GRID,),
    in_specs=[
        pl.BlockSpec((_ROWS_BLK, D_IN), lambda i: (i, 0)),
        pl.BlockSpec((D_IN, D_HID), lambda i: (0, 0)),
        pl.BlockSpec((_ROWS_BLK, 1), lambda i: (i, 0)),
    ],
    out_specs=[
        pl.BlockSpec((_ROWS_BLK, D_HID), lambda i: (i, 0)),
        pl.BlockSpec((_ROWS_BLK, 1), lambda i: (i, 0)),
    ],
    out_shape=[
        jax.ShapeDtypeStruct((N_NODES, D_HID), jnp.float32),
        jax.ShapeDtypeStruct((N_NODES, 1), jnp.float32),
    ],
)


def _final_body(acc_ref, s_ref, g_ref, b_ref, wf_ref, bf_ref, out_ref):
    i = pl.program_id(0)
    tot = acc_ref[0] + acc_ref[1] + s_ref[...]
    agg = g_ref[...] * tot
    o = jnp.maximum(agg + b_ref[...], 0.0)
    col = jnp.sum(o, axis=0, keepdims=True)
    part = jnp.sum(col * wf_ref[...], axis=1, keepdims=True)

    @pl.when(i == 0)
    def _init():
        out_ref[...] = jnp.zeros_like(out_ref)

    out_ref[...] += part

    @pl.when(i == _GRID - 1)
    def _finish():
        out_ref[...] = out_ref[...] / float(N_NODES) + bf_ref[...]


_final_call = pl.pallas_call(
    _final_body,
    grid=(_GRID,),
    in_specs=[
        pl.BlockSpec((NC, _ROWS_BLK, D_HID), lambda i: (0, i, 0)),
        pl.BlockSpec((_ROWS_BLK, D_HID), lambda i: (i, 0)),
        pl.BlockSpec((_ROWS_BLK, 1), lambda i: (i, 0)),
        pl.BlockSpec((1, D_HID), lambda i: (0, 0)),
        pl.BlockSpec((1, D_HID), lambda i: (0, 0)),
        pl.BlockSpec((1, 1), lambda i: (0, 0)),
    ],
    out_specs=pl.BlockSpec((1, 1), lambda i: (0, 0)),
    out_shape=jax.ShapeDtypeStruct((1, 1), jnp.float32),
)


@functools.cache
def _sc_kernels():
    mesh = plsc.VectorSubcoreMesh(core_axis_name="c", subcore_axis_name="s",
                                  num_cores=NC, num_subcores=NS)
    params = pltpu.CompilerParams(use_tc_tiling_on_sc=False)
    deg_kernel = pl.kernel(
        _deg_body,
        out_type=jax.ShapeDtypeStruct((NC, R_SH), jnp.float32),
        mesh=mesh,
        compiler_params=params,
        scratch_types=[
            pltpu.VMEM((NBLK, BLK), jnp.int32),
            pltpu.VMEM((BLK,), jnp.float32),
            pltpu.VMEM_SHARED((R_SH,), jnp.float32),
        ],
    )
    msg_kernel = pl.kernel(
        _msg_body,
        out_type=jax.ShapeDtypeStruct((NC, R_SH, D_HID), jnp.float32),
        mesh=mesh,
        compiler_params=params,
        scratch_types=[
            pltpu.VMEM((NBLK, BLK), jnp.int32),
            pltpu.VMEM((NBLK, BLK), jnp.int32),
            pltpu.VMEM((BLK, D_HID), jnp.float32),
            pltpu.VMEM((BLK, D_HID), jnp.float32),
            pltpu.VMEM_SHARED((R_SH, D_HID), jnp.float32),
            pltpu.SemaphoreType.DMA,
            pltpu.SemaphoreType.DMA,
        ],
    )
    return deg_kernel, msg_kernel


def kernel(x, edge_index, W, b, Wf, bf):
    deg_kernel, msg_kernel = _sc_kernels()
    ei = edge_index.astype(jnp.int32)
    src = ei[0].reshape(NW, EPT)
    dst = ei[1].reshape(NW, EPT)
    srcb = jnp.pad(src, ((0, 0), (0, PAD))).reshape(NW, NBLK, BLK)
    dstb = jnp.pad(dst, ((0, 0), (0, PAD)),
                   constant_values=N_NODES).reshape(NW, NBLK, BLK)
    zeros1 = jnp.zeros((R_SH,), jnp.float32)
    zeros2 = jnp.zeros((R_SH, D_HID), jnp.float32)

    degp = deg_kernel(dstb, zeros1)
    deg = (degp[0, :N_NODES] + degp[1, :N_NODES] + 1.0)[:, None]
    s_arr, g_arr = _feat_call(x, W, deg)
    accp = msg_kernel(srcb, dstb, s_arr, zeros2)
    res = _final_call(accp, s_arr, g_arr, b.reshape(1, D_HID),
                      Wf.reshape(1, D_HID), bf.reshape(1, 1))
    return res.reshape(1)


# speedup vs baseline: 60.4603x; 1.8719x over previous
import functools

import jax
import jax.numpy as jnp
from jax import lax
from jax.experimental import pallas as pl
from jax.experimental.pallas import tpu as pltpu
from jax.experimental.pallas import tpu_sc as plsc

N_NODES = 10000
N_EDGES = 320000
D_IN = 128
D_HID = 64

NC, NS = 2, 16
NW = NC * NS
BLK = 128
NBT = N_EDGES // BLK
NB = NBT // NW
NX = NBT - NB * NW
R_SH = 10240
RPT = R_SH // NS


def _deg_body(ei_hbm, zeros_hbm, out_hbm, idx_v, ones_v, deg_sh, sem):
    c = lax.axis_index("c")
    s = lax.axis_index("s")
    wid = s * NC + c
    has_extra = wid < NX
    pltpu.sync_copy(zeros_hbm.at[pl.ds(s * RPT, RPT)], deg_sh.at[pl.ds(s * RPT, RPT)])
    pltpu.sync_copy(ei_hbm.at[1, pl.ds(wid * NB, NB)], idx_v.at[pl.ds(0, NB)])

    @pl.when(has_extra)
    def _load_extra():
        pltpu.sync_copy(ei_hbm.at[1, NB * NW + wid], idx_v.at[NB])

    for i in range(BLK // 16):
        ones_v[pl.ds(i * 16, 16)] = jnp.ones((16,), jnp.float32)
    plsc.subcore_barrier()

    @pl.loop(0, NB)
    def _fire(j):
        pltpu.async_copy(ones_v, deg_sh.at[idx_v.at[j]], sem, add=True)

    @pl.when(has_extra)
    def _fire_extra():
        pltpu.async_copy(ones_v, deg_sh.at[idx_v.at[NB]], sem, add=True)

    @pl.loop(0, NB)
    def _drain(j):
        pltpu.make_async_copy(ones_v, deg_sh.at[idx_v.at[j]], sem).wait()

    @pl.when(has_extra)
    def _drain_extra():
        pltpu.make_async_copy(ones_v, deg_sh.at[idx_v.at[NB]], sem).wait()

    plsc.subcore_barrier()
    pltpu.sync_copy(deg_sh.at[pl.ds(s * RPT, RPT)],
                    out_hbm.at[c, pl.ds(s * RPT, RPT)])


def _msg_body(ei_hbm, s_hbm, zeros_hbm, out_hbm,
              si_v, di_v, r0, r1, r2, r3, acc_sh,
              g0, g1, g2, g3, s0, s1, s2, s3):
    c = lax.axis_index("c")
    s = lax.axis_index("s")
    wid = s * NC + c
    has_extra = wid < NX
    rows = (r0, r1, r2, r3)
    gsem = (g0, g1, g2, g3)
    ssem = (s0, s1, s2, s3)

    pltpu.sync_copy(zeros_hbm.at[pl.ds(s * RPT, RPT)], acc_sh.at[pl.ds(s * RPT, RPT)])
    pltpu.sync_copy(ei_hbm.at[0, pl.ds(wid * NB, NB)], si_v.at[pl.ds(0, NB)])
    pltpu.sync_copy(ei_hbm.at[1, pl.ds(wid * NB, NB)], di_v.at[pl.ds(0, NB)])

    @pl.when(has_extra)
    def _load_extra():
        pltpu.sync_copy(ei_hbm.at[0, NB * NW + wid], si_v.at[NB])
        pltpu.sync_copy(ei_hbm.at[1, NB * NW + wid], di_v.at[NB])

    plsc.subcore_barrier()

    def gather(j, b):
        pltpu.async_copy(s_hbm.at[si_v.at[j]], rows[b], gsem[b])

    def gather_wait(j, b):
        pltpu.make_async_copy(s_hbm.at[si_v.at[j]], rows[b], gsem[b]).wait()

    def scatter(j, b):
        pltpu.async_copy(rows[b], acc_sh.at[di_v.at[j]], ssem[b], add=True)

    def scatter_wait(j, b):
        pltpu.make_async_copy(rows[b], acc_sh.at[di_v.at[j]], ssem[b]).wait()

    gather(0, 0)
    gather(1, 1)
    gather(2, 2)
    gather_wait(0, 0)
    scatter(0, 0)
    gather(3, 3)
    gather_wait(1, 1)
    scatter(1, 1)

    @pl.loop(0, (NB - 6) // 4)
    def _steady(q):
        base = 2 + q * 4
        for k in range(4):
            j = base + k
            b = (2 + k) % 4
            bn = k
            scatter_wait(j - 2, bn)
            gather(j + 2, bn)
            gather_wait(j, b)
            scatter(j, b)

    for jj in (NB - 4, NB - 3):
        bn = (jj + 2) % 4
        scatter_wait(jj - 2, bn)
        gather(jj + 2, bn)
        gather_wait(jj, jj % 4)
        scatter(jj, jj % 4)
    for jj in (NB - 2, NB - 1):
        gather_wait(jj, jj % 4)
        scatter(jj, jj % 4)

    @pl.when(has_extra)
    def _extra():
        scatter_wait(NB - 4, 2)
        gather(NB, 2)
        gather_wait(NB, 2)
        scatter(NB, 2)

    scatter_wait(NB - 1, 3)
    scatter_wait(NB - 2, 0)
    scatter_wait(NB - 1, 1)
    scatter_wait(NB - 4, 2)

    plsc.subcore_barrier()
    pltpu.sync_copy(acc_sh.at[pl.ds(s * RPT, RPT)],
                    out_hbm.at[c, pl.ds(s * RPT, RPT)])


_ROWS_BLK = 1000
_GRID = N_NODES // _ROWS_BLK


def _feat_body(x_ref, w_ref, deg_ref, s_ref, g_ref):
    g = lax.rsqrt(deg_ref[...])
    h = jnp.dot(x_ref[...], w_ref[...], preferred_element_type=jnp.float32)
    s_ref[...] = h * g
    g_ref[...] = g


_feat_call = pl.pallas_call(
    _feat_body,
    grid=(_GRID,),
    in_specs=[
        pl.BlockSpec((_ROWS_BLK, D_IN), lambda i: (i, 0)),
        pl.BlockSpec((D_IN, D_HID), lambda i: (0, 0)),
        pl.BlockSpec((_ROWS_BLK, 1), lambda i: (i, 0)),
    ],
    out_specs=[
        pl.BlockSpec((_ROWS_BLK, D_HID), lambda i: (i, 0)),
        pl.BlockSpec((_ROWS_BLK, 1), lambda i: (i, 0)),
    ],
    out_shape=[
        jax.ShapeDtypeStruct((N_NODES, D_HID), jnp.float32),
        jax.ShapeDtypeStruct((N_NODES, 1), jnp.float32),
    ],
)


def _final_body(acc_ref, s_ref, g_ref, b_ref, wf_ref, bf_ref, out_ref):
    i = pl.program_id(0)
    tot = acc_ref[0] + acc_ref[1] + s_ref[...]
    agg = g_ref[...] * tot
    o = jnp.maximum(agg + b_ref[...], 0.0)
    o_r = o.astype(jnp.bfloat16).astype(jnp.float32)
    wf_r = wf_ref[...].astype(jnp.bfloat16).astype(jnp.float32)
    col = jnp.sum(o_r, axis=0, keepdims=True)
    part = jnp.sum(col * wf_r, axis=1, keepdims=True)

    @pl.when(i == 0)
    def _init():
        out_ref[...] = jnp.zeros_like(out_ref)

    out_ref[...] += part

    @pl.when(i == _GRID - 1)
    def _finish():
        out_ref[...] = out_ref[...] / float(N_NODES) + bf_ref[...]


_final_call = pl.pallas_call(
    _final_body,
    grid=(_GRID,),
    in_specs=[
        pl.BlockSpec((NC, _ROWS_BLK, D_HID), lambda i: (0, i, 0)),
        pl.BlockSpec((_ROWS_BLK, D_HID), lambda i: (i, 0)),
        pl.BlockSpec((_ROWS_BLK, 1), lambda i: (i, 0)),
        pl.BlockSpec((1, D_HID), lambda i: (0, 0)),
        pl.BlockSpec((1, D_HID), lambda i: (0, 0)),
        pl.BlockSpec((1, 1), lambda i: (0, 0)),
    ],
    out_specs=pl.BlockSpec((1, 1), lambda i: (0, 0)),
    out_shape=jax.ShapeDtypeStruct((1, 1), jnp.float32),
)


@functools.cache
def _sc_kernels():
    mesh = plsc.VectorSubcoreMesh(core_axis_name="c", subcore_axis_name="s",
                                  num_cores=NC, num_subcores=NS)
    params = pltpu.CompilerParams(use_tc_tiling_on_sc=False)
    deg_kernel = pl.kernel(
        _deg_body,
        out_type=jax.ShapeDtypeStruct((NC, R_SH), jnp.float32),
        mesh=mesh,
        compiler_params=params,
        scratch_types=[
            pltpu.VMEM((NB + 1, BLK), jnp.int32),
            pltpu.VMEM((BLK,), jnp.float32),
            pltpu.VMEM_SHARED((R_SH,), jnp.float32),
            pltpu.SemaphoreType.DMA,
        ],
    )
    msg_kernel = pl.kernel(
        _msg_body,
        out_type=jax.ShapeDtypeStruct((NC, R_SH, D_HID), jnp.float32),
        mesh=mesh,
        compiler_params=params,
        scratch_types=[
            pltpu.VMEM((NB + 1, BLK), jnp.int32),
            pltpu.VMEM((NB + 1, BLK), jnp.int32),
            pltpu.VMEM((BLK, D_HID), jnp.float32),
            pltpu.VMEM((BLK, D_HID), jnp.float32),
            pltpu.VMEM((BLK, D_HID), jnp.float32),
            pltpu.VMEM((BLK, D_HID), jnp.float32),
            pltpu.VMEM_SHARED((R_SH, D_HID), jnp.float32),
            pltpu.SemaphoreType.DMA,
            pltpu.SemaphoreType.DMA,
            pltpu.SemaphoreType.DMA,
            pltpu.SemaphoreType.DMA,
            pltpu.SemaphoreType.DMA,
            pltpu.SemaphoreType.DMA,
            pltpu.SemaphoreType.DMA,
            pltpu.SemaphoreType.DMA,
        ],
    )
    return deg_kernel, msg_kernel


def kernel(x, edge_index, W, b, Wf, bf):
    deg_kernel, msg_kernel = _sc_kernels()
    ei3 = edge_index.astype(jnp.int32).reshape(2, NBT, BLK)
    zeros1 = jnp.zeros((R_SH,), jnp.float32)
    zeros2 = jnp.zeros((R_SH, D_HID), jnp.float32)

    degp = deg_kernel(ei3, zeros1)
    deg = (degp[0, :N_NODES] + degp[1, :N_NODES] + 1.0)[:, None]
    s_arr, g_arr = _feat_call(x, W, deg)
    accp = msg_kernel(ei3, s_arr, zeros2)
    res = _final_call(accp, s_arr, g_arr, b.reshape(1, D_HID),
                      Wf.reshape(1, D_HID), bf.reshape(1, 1))
    return res.reshape(1)


# speedup vs baseline: 68.6965x; 1.1362x over previous
import functools

import jax
import jax.numpy as jnp
from jax import lax
from jax.experimental import pallas as pl
from jax.experimental.pallas import tpu as pltpu
from jax.experimental.pallas import tpu_sc as plsc

N_NODES = 10000
N_EDGES = 320000
D_IN = 128
D_HID = 64

NC, NS = 2, 16
NW = NC * NS
BLK = 128
NBS = -(-N_NODES // BLK)
SPAD = NBS * BLK - N_NODES
NBT = N_EDGES // BLK + NBS
NB = NBT // NW
NX = NBT - NB * NW
R_SH = 10240
RPT = R_SH // NS


def _deg_body(ei_hbm, zeros_hbm, out_hbm, idx_v, ones_v, deg_sh, sem):
    c = lax.axis_index("c")
    s = lax.axis_index("s")
    wid = s * NC + c
    has_extra = wid < NX
    pltpu.sync_copy(zeros_hbm.at[pl.ds(s * RPT, RPT)], deg_sh.at[pl.ds(s * RPT, RPT)])
    pltpu.sync_copy(ei_hbm.at[1, pl.ds(wid * NB, NB)], idx_v.at[pl.ds(0, NB)])

    @pl.when(has_extra)
    def _load_extra():
        pltpu.sync_copy(ei_hbm.at[1, NB * NW + wid], idx_v.at[NB])

    for i in range(BLK // 16):
        ones_v[pl.ds(i * 16, 16)] = jnp.ones((16,), jnp.float32)
    plsc.subcore_barrier()

    @pl.loop(0, NB)
    def _fire(j):
        pltpu.async_copy(ones_v, deg_sh.at[idx_v.at[j]], sem, add=True)

    @pl.when(has_extra)
    def _fire_extra():
        pltpu.async_copy(ones_v, deg_sh.at[idx_v.at[NB]], sem, add=True)

    @pl.loop(0, NB)
    def _drain(j):
        pltpu.make_async_copy(ones_v, deg_sh.at[idx_v.at[j]], sem).wait()

    @pl.when(has_extra)
    def _drain_extra():
        pltpu.make_async_copy(ones_v, deg_sh.at[idx_v.at[NB]], sem).wait()

    plsc.subcore_barrier()
    pltpu.sync_copy(deg_sh.at[pl.ds(s * RPT, RPT)],
                    out_hbm.at[c, pl.ds(s * RPT, RPT)])


def _msg_body(ei_hbm, s_hbm, zeros_hbm, out_hbm,
              si_v, di_v, r0, r1, r2, r3, acc_sh,
              g0, g1, g2, g3, s0, s1, s2, s3):
    c = lax.axis_index("c")
    s = lax.axis_index("s")
    wid = s * NC + c
    has_extra = wid < NX
    rows = (r0, r1, r2, r3)
    gsem = (g0, g1, g2, g3)
    ssem = (s0, s1, s2, s3)

    pltpu.sync_copy(zeros_hbm.at[pl.ds(s * RPT, RPT)], acc_sh.at[pl.ds(s * RPT, RPT)])
    pltpu.sync_copy(ei_hbm.at[0, pl.ds(wid * NB, NB)], si_v.at[pl.ds(0, NB)])
    pltpu.sync_copy(ei_hbm.at[1, pl.ds(wid * NB, NB)], di_v.at[pl.ds(0, NB)])

    @pl.when(has_extra)
    def _load_extra():
        pltpu.sync_copy(ei_hbm.at[0, NB * NW + wid], si_v.at[NB])
        pltpu.sync_copy(ei_hbm.at[1, NB * NW + wid], di_v.at[NB])

    plsc.subcore_barrier()

    def gather(j, b):
        pltpu.async_copy(s_hbm.at[si_v.at[j]], rows[b], gsem[b])

    def gather_wait(j, b):
        pltpu.make_async_copy(s_hbm.at[si_v.at[j]], rows[b], gsem[b]).wait()

    def scatter(j, b):
        pltpu.async_copy(rows[b], acc_sh.at[di_v.at[j]], ssem[b], add=True)

    def scatter_wait(j, b):
        pltpu.make_async_copy(rows[b], acc_sh.at[di_v.at[j]], ssem[b]).wait()

    _Q = (NB - 6) // 4
    gather(0, 0)
    gather(1, 1)
    gather(2, 2)
    gather_wait(0, 0)
    scatter(0, 0)
    gather(3, 3)
    gather_wait(1, 1)
    scatter(1, 1)

    @pl.loop(0, _Q)
    def _steady(q):
        base = 2 + q * 4
        for k in range(4):
            j = base + k
            b = (2 + k) % 4
            scatter_wait(j - 2, k)
            gather(j + 2, k)
            gather_wait(j, b)
            scatter(j, b)

    for jj in range(2 + 4 * _Q, NB):
        b = jj % 4
        if jj + 2 < NB:
            bn = (jj + 2) % 4
            scatter_wait(jj - 2, bn)
            gather(jj + 2, bn)
        gather_wait(jj, b)
        scatter(jj, b)

    @pl.when(has_extra)
    def _extra():
        scatter_wait(NB - 4, NB % 4)
        gather(NB, NB % 4)
        gather_wait(NB, NB % 4)
        scatter(NB, NB % 4)

    for b in range(4):
        scatter_wait(NB - 4 + b, (NB - 4 + b) % 4)

    plsc.subcore_barrier()
    pltpu.sync_copy(acc_sh.at[pl.ds(s * RPT, RPT)],
                    out_hbm.at[c, pl.ds(s * RPT, RPT)])


_ROWS_BLK = 2000
_GRID = N_NODES // _ROWS_BLK
_PROWS = R_SH // 2 // _GRID
_NPAIR = N_NODES // 2


def _feat_body(x_ref, w_ref, deg_ref, s_ref):
    g = lax.rsqrt(deg_ref[...].astype(jnp.float32))
    h = jnp.dot(x_ref[...], w_ref[...], preferred_element_type=jnp.float32)
    s_ref[...] = h * g


_feat_call = pl.pallas_call(
    _feat_body,
    grid=(_GRID,),
    in_specs=[
        pl.BlockSpec((_ROWS_BLK, D_IN), lambda i: (i, 0)),
        pl.BlockSpec((D_IN, D_HID), lambda i: (0, 0)),
        pl.BlockSpec((_ROWS_BLK, 1), lambda i: (i, 0)),
    ],
    out_specs=pl.BlockSpec((_ROWS_BLK, D_HID), lambda i: (i, 0)),
    out_shape=jax.ShapeDtypeStruct((N_NODES, D_HID), jnp.float32),
)


def _final_body(acc_ref, deg_ref, b_ref, wf_ref, bf_ref, out_ref):
    i = pl.program_id(0)
    g = lax.rsqrt(deg_ref[...].astype(jnp.float32))
    agg = g * (acc_ref[0] + acc_ref[1])
    o = jnp.maximum(agg + b_ref[...], 0.0)
    p = lax.broadcasted_iota(jnp.int32, (_PROWS, 2 * D_HID), 0) + i * _PROWS
    o = jnp.where(p < _NPAIR, o, 0.0)
    o_r = o.astype(jnp.bfloat16).astype(jnp.float32)
    wf_r = wf_ref[...].astype(jnp.bfloat16).astype(jnp.float32)
    col = jnp.sum(o_r, axis=0, keepdims=True)
    part = jnp.sum(col * wf_r, axis=1, keepdims=True)

    @pl.when(i == 0)
    def _init():
        out_ref[...] = jnp.zeros_like(out_ref)

    out_ref[...] += part

    @pl.when(i == _GRID - 1)
    def _finish():
        out_ref[...] = out_ref[...] / float(N_NODES) + bf_ref[...]


_final_call = pl.pallas_call(
    _final_body,
    grid=(_GRID,),
    in_specs=[
        pl.BlockSpec((NC, _PROWS, 2 * D_HID), lambda i: (0, i, 0)),
        pl.BlockSpec((_PROWS, 2 * D_HID), lambda i: (i, 0)),
        pl.BlockSpec((1, 2 * D_HID), lambda i: (0, 0)),
        pl.BlockSpec((1, 2 * D_HID), lambda i: (0, 0)),
        pl.BlockSpec((1, 1), lambda i: (0, 0)),
    ],
    out_specs=pl.BlockSpec((1, 1), lambda i: (0, 0)),
    out_shape=jax.ShapeDtypeStruct((1, 1), jnp.float32),
)


@functools.cache
def _sc_kernels():
    mesh = plsc.VectorSubcoreMesh(core_axis_name="c", subcore_axis_name="s",
                                  num_cores=NC, num_subcores=NS)
    params = pltpu.CompilerParams(use_tc_tiling_on_sc=False)
    deg_kernel = pl.kernel(
        _deg_body,
        out_type=jax.ShapeDtypeStruct((NC, R_SH), jnp.float32),
        mesh=mesh,
        compiler_params=params,
        scratch_types=[
            pltpu.VMEM((NB + 1, BLK), jnp.int32),
            pltpu.VMEM((BLK,), jnp.float32),
            pltpu.VMEM_SHARED((R_SH,), jnp.float32),
            pltpu.SemaphoreType.DMA,
        ],
    )
    msg_kernel = pl.kernel(
        _msg_body,
        out_type=jax.ShapeDtypeStruct((NC, R_SH, D_HID), jnp.float32),
        mesh=mesh,
        compiler_params=params,
        scratch_types=[
            pltpu.VMEM((NB + 1, BLK), jnp.int32),
            pltpu.VMEM((NB + 1, BLK), jnp.int32),
            pltpu.VMEM((BLK, D_HID), jnp.float32),
            pltpu.VMEM((BLK, D_HID), jnp.float32),
            pltpu.VMEM((BLK, D_HID), jnp.float32),
            pltpu.VMEM((BLK, D_HID), jnp.float32),
            pltpu.VMEM_SHARED((R_SH, D_HID), jnp.float32),
            pltpu.SemaphoreType.DMA,
            pltpu.SemaphoreType.DMA,
            pltpu.SemaphoreType.DMA,
            pltpu.SemaphoreType.DMA,
            pltpu.SemaphoreType.DMA,
            pltpu.SemaphoreType.DMA,
            pltpu.SemaphoreType.DMA,
            pltpu.SemaphoreType.DMA,
        ],
    )
    return deg_kernel, msg_kernel


def kernel(x, edge_index, W, b, Wf, bf):
    deg_kernel, msg_kernel = _sc_kernels()
    ei = edge_index.astype(jnp.int32)
    sl_src = jnp.concatenate(
        [jnp.arange(N_NODES, dtype=jnp.int32),
         jnp.zeros((SPAD,), jnp.int32)]).reshape(NBS, BLK)
    sl_dst = jnp.concatenate(
        [jnp.arange(N_NODES, dtype=jnp.int32),
         jnp.full((SPAD,), N_NODES, jnp.int32)]).reshape(NBS, BLK)
    ei3 = jnp.concatenate(
        [ei.reshape(2, N_EDGES // BLK, BLK),
         jnp.stack([sl_src, sl_dst])], axis=1)
    zeros1 = jnp.zeros((R_SH,), jnp.float32)
    zeros2 = jnp.zeros((R_SH, D_HID), jnp.float32)

    degp = deg_kernel(ei3, zeros1)
    degt = degp[0] + degp[1]
    degc = degt[:N_NODES, None].astype(jnp.bfloat16)
    degc128 = jnp.broadcast_to(
        degt.reshape(R_SH // 2, 2, 1),
        (R_SH // 2, 2, D_HID)).reshape(R_SH // 2, 2 * D_HID).astype(jnp.bfloat16)

    s_arr = _feat_call(x, W, degc)
    accp = msg_kernel(ei3, s_arr, zeros2)
    accp2 = accp.reshape(NC, R_SH // 2, 2 * D_HID)

    b128 = jnp.concatenate([b, b]).reshape(1, 2 * D_HID)
    wf128 = jnp.concatenate([Wf[:, 0], Wf[:, 0]]).reshape(1, 2 * D_HID)
    res = _final_call(accp2, degc128, b128, wf128, bf.reshape(1, 1))
    return res.reshape(1)


# speedup vs baseline: 70.1941x; 1.0218x over previous
import functools

import jax
import jax.numpy as jnp
from jax import lax
from jax.experimental import pallas as pl
from jax.experimental.pallas import tpu as pltpu
from jax.experimental.pallas import tpu_sc as plsc

N_NODES = 10000
N_EDGES = 320000
D_IN = 128
D_HID = 64

NC, NS = 2, 16
NW = NC * NS
BLK = 128
NBS = -(-N_NODES // BLK)
SPAD = NBS * BLK - N_NODES
NBT = N_EDGES // BLK + NBS
NB = NBT // NW
NX = NBT - NB * NW
R_SH = 10240
RPT = R_SH // NS


def _deg_body(ei_hbm, out_hbm, idx_v, ones_v, zbuf, deg_sh, sem):
    c = lax.axis_index("c")
    s = lax.axis_index("s")
    wid = s * NC + c
    has_extra = wid < NX
    for i in range(RPT // 16):
        zbuf[pl.ds(i * 16, 16)] = jnp.zeros((16,), jnp.float32)
    pltpu.sync_copy(zbuf, deg_sh.at[pl.ds(s * RPT, RPT)])
    pltpu.sync_copy(ei_hbm.at[1, pl.ds(wid * NB, NB)], idx_v.at[pl.ds(0, NB)])

    @pl.when(has_extra)
    def _load_extra():
        pltpu.sync_copy(ei_hbm.at[1, NB * NW + wid], idx_v.at[NB])

    for i in range(BLK // 16):
        ones_v[pl.ds(i * 16, 16)] = jnp.ones((16,), jnp.float32)
    plsc.subcore_barrier()

    @pl.loop(0, NB)
    def _fire(j):
        pltpu.async_copy(ones_v, deg_sh.at[idx_v.at[j]], sem, add=True)

    @pl.when(has_extra)
    def _fire_extra():
        pltpu.async_copy(ones_v, deg_sh.at[idx_v.at[NB]], sem, add=True)

    @pl.loop(0, NB)
    def _drain(j):
        pltpu.make_async_copy(ones_v, deg_sh.at[idx_v.at[j]], sem).wait()

    @pl.when(has_extra)
    def _drain_extra():
        pltpu.make_async_copy(ones_v, deg_sh.at[idx_v.at[NB]], sem).wait()

    plsc.subcore_barrier()
    pltpu.sync_copy(deg_sh.at[pl.ds(s * RPT, RPT)],
                    out_hbm.at[c, pl.ds(s * RPT, RPT)])


def _msg_body(ei_hbm, s_hbm, out_hbm,
              si_v, di_v, r0, r1, r2, r3, acc_sh,
              g0, g1, g2, g3, s0, s1, s2, s3):
    c = lax.axis_index("c")
    s = lax.axis_index("s")
    wid = s * NC + c
    has_extra = wid < NX
    rows = (r0, r1, r2, r3)
    gsem = (g0, g1, g2, g3)
    ssem = (s0, s1, s2, s3)

    for rr in range(BLK):
        for cc in range(D_HID // 16):
            r0[rr, pl.ds(cc * 16, 16)] = jnp.zeros((16,), jnp.float32)
    for k in range(RPT // BLK):
        pltpu.sync_copy(r0, acc_sh.at[pl.ds(s * RPT + k * BLK, BLK)])
    pltpu.sync_copy(ei_hbm.at[0, pl.ds(wid * NB, NB)], si_v.at[pl.ds(0, NB)])
    pltpu.sync_copy(ei_hbm.at[1, pl.ds(wid * NB, NB)], di_v.at[pl.ds(0, NB)])

    @pl.when(has_extra)
    def _load_extra():
        pltpu.sync_copy(ei_hbm.at[0, NB * NW + wid], si_v.at[NB])
        pltpu.sync_copy(ei_hbm.at[1, NB * NW + wid], di_v.at[NB])

    plsc.subcore_barrier()

    def gather(j, b):
        pltpu.async_copy(s_hbm.at[si_v.at[j]], rows[b], gsem[b])

    def gather_wait(j, b):
        pltpu.make_async_copy(s_hbm.at[si_v.at[j]], rows[b], gsem[b]).wait()

    def scatter(j, b):
        pltpu.async_copy(rows[b], acc_sh.at[di_v.at[j]], ssem[b], add=True)

    def scatter_wait(j, b):
        pltpu.make_async_copy(rows[b], acc_sh.at[di_v.at[j]], ssem[b]).wait()

    _Q = (NB - 6) // 4
    gather(0, 0)
    gather(1, 1)
    gather(2, 2)
    gather_wait(0, 0)
    scatter(0, 0)
    gather(3, 3)
    gather_wait(1, 1)
    scatter(1, 1)

    @pl.loop(0, _Q)
    def _steady(q):
        base = 2 + q * 4
        for k in range(4):
            j = base + k
            b = (2 + k) % 4
            scatter_wait(j - 2, k)
            gather(j + 2, k)
            gather_wait(j, b)
            scatter(j, b)

    for jj in range(2 + 4 * _Q, NB):
        b = jj % 4
        if jj + 2 < NB:
            bn = (jj + 2) % 4
            scatter_wait(jj - 2, bn)
            gather(jj + 2, bn)
        gather_wait(jj, b)
        scatter(jj, b)

    @pl.when(has_extra)
    def _extra():
        scatter_wait(NB - 4, NB % 4)
        gather(NB, NB % 4)
        gather_wait(NB, NB % 4)
        scatter(NB, NB % 4)

    for b in range(4):
        scatter_wait(NB - 4 + b, (NB - 4 + b) % 4)

    plsc.subcore_barrier()
    pltpu.sync_copy(acc_sh.at[pl.ds(s * RPT, RPT)],
                    out_hbm.at[c, pl.ds(s * RPT, RPT)])


_ROWS_BLK = 2000
_GRID = N_NODES // _ROWS_BLK
_PROWS = R_SH // 2 // _GRID
_NPAIR = N_NODES // 2


def _h_body(x_ref, w_ref, h_ref):
    h_ref[...] = jnp.dot(x_ref[...], w_ref[...],
                         preferred_element_type=jnp.float32)


_h_call = pl.pallas_call(
    _h_body,
    grid=(_GRID,),
    in_specs=[
        pl.BlockSpec((_ROWS_BLK, D_IN), lambda i: (i, 0)),
        pl.BlockSpec((D_IN, D_HID), lambda i: (0, 0)),
    ],
    out_specs=pl.BlockSpec((_ROWS_BLK, D_HID), lambda i: (i, 0)),
    out_shape=jax.ShapeDtypeStruct((N_NODES, D_HID), jnp.float32),
)


def _scale_body(h_ref, deg_ref, s_ref):
    g = lax.rsqrt(deg_ref[...].astype(jnp.float32))
    s_ref[...] = h_ref[...] * g


_scale_call = pl.pallas_call(
    _scale_body,
    grid=(_GRID,),
    in_specs=[
        pl.BlockSpec((_ROWS_BLK, D_HID), lambda i: (i, 0)),
        pl.BlockSpec((_ROWS_BLK, 1), lambda i: (i, 0)),
    ],
    out_specs=pl.BlockSpec((_ROWS_BLK, D_HID), lambda i: (i, 0)),
    out_shape=jax.ShapeDtypeStruct((N_NODES, D_HID), jnp.float32),
)


def _final_body(acc_ref, deg_ref, b_ref, wf_ref, bf_ref, out_ref):
    i = pl.program_id(0)
    g = lax.rsqrt(deg_ref[...].astype(jnp.float32))
    agg = g * (acc_ref[0] + acc_ref[1])
    o = jnp.maximum(agg + b_ref[...], 0.0)
    p = lax.broadcasted_iota(jnp.int32, (_PROWS, 2 * D_HID), 0) + i * _PROWS
    o = jnp.where(p < _NPAIR, o, 0.0)
    o_r = o.astype(jnp.bfloat16).astype(jnp.float32)
    wf_r = wf_ref[...].astype(jnp.bfloat16).astype(jnp.float32)
    col = jnp.sum(o_r, axis=0, keepdims=True)
    part = jnp.sum(col * wf_r, axis=1, keepdims=True)

    @pl.when(i == 0)
    def _init():
        out_ref[...] = jnp.zeros_like(out_ref)

    out_ref[...] += part

    @pl.when(i == _GRID - 1)
    def _finish():
        out_ref[...] = out_ref[...] / float(N_NODES) + bf_ref[...]


_final_call = pl.pallas_call(
    _final_body,
    grid=(_GRID,),
    in_specs=[
        pl.BlockSpec((NC, _PROWS, 2 * D_HID), lambda i: (0, i, 0)),
        pl.BlockSpec((_PROWS, 2 * D_HID), lambda i: (i, 0)),
        pl.BlockSpec((1, 2 * D_HID), lambda i: (0, 0)),
        pl.BlockSpec((1, 2 * D_HID), lambda i: (0, 0)),
        pl.BlockSpec((1, 1), lambda i: (0, 0)),
    ],
    out_specs=pl.BlockSpec((1, 1), lambda i: (0, 0)),
    out_shape=jax.ShapeDtypeStruct((1, 1), jnp.float32),
)


@functools.cache
def _sc_kernels():
    mesh = plsc.VectorSubcoreMesh(core_axis_name="c", subcore_axis_name="s",
                                  num_cores=NC, num_subcores=NS)
    params = pltpu.CompilerParams(use_tc_tiling_on_sc=False)
    deg_kernel = pl.kernel(
        _deg_body,
        out_type=jax.ShapeDtypeStruct((NC, R_SH), jnp.float32),
        mesh=mesh,
        compiler_params=params,
        scratch_types=[
            pltpu.VMEM((NB + 1, BLK), jnp.int32),
            pltpu.VMEM((BLK,), jnp.float32),
            pltpu.VMEM((RPT,), jnp.float32),
            pltpu.VMEM_SHARED((R_SH,), jnp.float32),
            pltpu.SemaphoreType.DMA,
        ],
    )
    msg_kernel = pl.kernel(
        _msg_body,
        out_type=jax.ShapeDtypeStruct((NC, R_SH, D_HID), jnp.float32),
        mesh=mesh,
        compiler_params=params,
        scratch_types=[
            pltpu.VMEM((NB + 1, BLK), jnp.int32),
            pltpu.VMEM((NB + 1, BLK), jnp.int32),
            pltpu.VMEM((BLK, D_HID), jnp.float32),
            pltpu.VMEM((BLK, D_HID), jnp.float32),
            pltpu.VMEM((BLK, D_HID), jnp.float32),
            pltpu.VMEM((BLK, D_HID), jnp.float32),
            pltpu.VMEM_SHARED((R_SH, D_HID), jnp.float32),
            pltpu.SemaphoreType.DMA,
            pltpu.SemaphoreType.DMA,
            pltpu.SemaphoreType.DMA,
            pltpu.SemaphoreType.DMA,
            pltpu.SemaphoreType.DMA,
            pltpu.SemaphoreType.DMA,
            pltpu.SemaphoreType.DMA,
            pltpu.SemaphoreType.DMA,
        ],
    )
    return deg_kernel, msg_kernel


def kernel(x, edge_index, W, b, Wf, bf):
    deg_kernel, msg_kernel = _sc_kernels()
    ei = edge_index.astype(jnp.int32)
    sl_src = jnp.concatenate(
        [jnp.arange(N_NODES, dtype=jnp.int32),
         jnp.zeros((SPAD,), jnp.int32)]).reshape(NBS, BLK)
    sl_dst = jnp.concatenate(
        [jnp.arange(N_NODES, dtype=jnp.int32),
         jnp.full((SPAD,), N_NODES, jnp.int32)]).reshape(NBS, BLK)
    ei3 = jnp.concatenate(
        [ei.reshape(2, N_EDGES // BLK, BLK),
         jnp.stack([sl_src, sl_dst])], axis=1)
    h_arr = _h_call(x, W)
    degp = deg_kernel(ei3)
    degt = degp[0] + degp[1]
    degc = degt[:N_NODES, None].astype(jnp.bfloat16)
    degc128 = jnp.broadcast_to(
        degt.reshape(R_SH // 2, 2, 1),
        (R_SH // 2, 2, D_HID)).reshape(R_SH // 2, 2 * D_HID).astype(jnp.bfloat16)

    s_arr = _scale_call(h_arr, degc)
    accp = msg_kernel(ei3, s_arr)
    accp2 = accp.reshape(NC, R_SH // 2, 2 * D_HID)

    b128 = jnp.concatenate([b, b]).reshape(1, 2 * D_HID)
    wf128 = jnp.concatenate([Wf[:, 0], Wf[:, 0]]).reshape(1, 2 * D_HID)
    res = _final_call(accp2, degc128, b128, wf128, bf.reshape(1, 1))
    return res.reshape(1)


# speedup vs baseline: 74.4627x; 1.0608x over previous
import functools

import jax
import jax.numpy as jnp
from jax import lax
from jax.experimental import pallas as pl
from jax.experimental.pallas import tpu as pltpu
from jax.experimental.pallas import tpu_sc as plsc

N_NODES = 10000
N_EDGES = 320000
D_IN = 128
D_HID = 64

NC, NS = 2, 16
NW = NC * NS
BLK = 128
NBS = -(-N_NODES // BLK)
SPAD = NBS * BLK - N_NODES
NBT = N_EDGES // BLK + NBS
NB = NBT // NW
NX = NBT - NB * NW
R_SH = 10240
RPT = R_SH // NS


def _deg_body(ei_hbm, out_hbm, idx_v, ones_v, zbuf, deg_sh, sem):
    c = lax.axis_index("c")
    s = lax.axis_index("s")
    wid = s * NC + c
    has_extra = wid < NX
    for i in range(RPT // 16):
        zbuf[pl.ds(i * 16, 16)] = jnp.zeros((16,), jnp.float32)
    pltpu.sync_copy(zbuf, deg_sh.at[pl.ds(s * RPT, RPT)])
    pltpu.sync_copy(ei_hbm.at[1, pl.ds(wid * NB, NB)], idx_v.at[pl.ds(0, NB)])

    @pl.when(has_extra)
    def _load_extra():
        pltpu.sync_copy(ei_hbm.at[1, NB * NW + wid], idx_v.at[NB])

    for i in range(BLK // 16):
        ones_v[pl.ds(i * 16, 16)] = jnp.ones((16,), jnp.float32)
    plsc.subcore_barrier()

    @pl.loop(0, NB)
    def _fire(j):
        pltpu.async_copy(ones_v, deg_sh.at[idx_v.at[j]], sem, add=True)

    @pl.when(has_extra)
    def _fire_extra():
        pltpu.async_copy(ones_v, deg_sh.at[idx_v.at[NB]], sem, add=True)

    @pl.loop(0, NB)
    def _drain(j):
        pltpu.make_async_copy(ones_v, deg_sh.at[idx_v.at[j]], sem).wait()

    @pl.when(has_extra)
    def _drain_extra():
        pltpu.make_async_copy(ones_v, deg_sh.at[idx_v.at[NB]], sem).wait()

    plsc.subcore_barrier()
    pltpu.sync_copy(deg_sh.at[pl.ds(s * RPT, RPT)],
                    out_hbm.at[c, pl.ds(s * RPT, RPT)])


_NBUF = 6


def _msg_body(ei_hbm, s_hbm, out_hbm,
              si_v, di_v, r0, r1, r2, r3, r4, r5, zbuf, acc_sh,
              g0, g1, g2, g3, g4, g5, s0, s1, s2, s3, s4, s5):
    c = lax.axis_index("c")
    s = lax.axis_index("s")
    wid = s * NC + c
    has_extra = wid < NX
    rows = (r0, r1, r2, r3, r4, r5)
    gsem = (g0, g1, g2, g3, g4, g5)
    ssem = (s0, s1, s2, s3, s4, s5)
    L = _NBUF // 2

    pltpu.sync_copy(ei_hbm.at[0, pl.ds(wid * NB, NB)], si_v.at[pl.ds(0, NB)])
    pltpu.sync_copy(ei_hbm.at[1, pl.ds(wid * NB, NB)], di_v.at[pl.ds(0, NB)])

    @pl.when(has_extra)
    def _load_extra():
        pltpu.sync_copy(ei_hbm.at[0, NB * NW + wid], si_v.at[NB])
        pltpu.sync_copy(ei_hbm.at[1, NB * NW + wid], di_v.at[NB])

    def gather(j, b):
        pltpu.async_copy(s_hbm.at[si_v.at[j]], rows[b], gsem[b])

    def gather_wait(j, b):
        pltpu.make_async_copy(s_hbm.at[si_v.at[j]], rows[b], gsem[b]).wait()

    def scatter(j, b):
        pltpu.async_copy(rows[b], acc_sh.at[di_v.at[j]], ssem[b], add=True)

    def scatter_wait(j, b):
        pltpu.make_async_copy(rows[b], acc_sh.at[di_v.at[j]], ssem[b]).wait()

    for j in range(L):
        gather(j, j)
    for cc in range(D_HID // 16):
        zbuf[0, pl.ds(cc * 16, 16)] = jnp.zeros((16,), jnp.float32)
    for rr in range(1, BLK):
        for cc in range(D_HID // 16):
            zbuf[rr, pl.ds(cc * 16, 16)] = jnp.zeros((16,), jnp.float32)
    for k in range(RPT // BLK):
        pltpu.sync_copy(zbuf, acc_sh.at[pl.ds(s * RPT + k * BLK, BLK)])
    plsc.subcore_barrier()

    for j in range(L):
        gather(j + L, j + L)
        gather_wait(j, j)
        scatter(j, j)

    _Q = (NB - L - 5) // _NBUF
    @pl.loop(0, _Q)
    def _steady(q):
        base = L + q * _NBUF
        for k in range(_NBUF):
            j = base + k
            b = (L + k) % _NBUF
            scatter_wait(j - L, k)
            gather(j + L, k)
            gather_wait(j, b)
            scatter(j, b)

    for jj in range(L + _NBUF * _Q, NB):
        b = jj % _NBUF
        if jj + L < NB:
            bn = (jj + L) % _NBUF
            scatter_wait(jj - L, bn)
            gather(jj + L, bn)
        gather_wait(jj, b)
        scatter(jj, b)

    @pl.when(has_extra)
    def _extra():
        scatter_wait(NB - _NBUF, NB % _NBUF)
        gather(NB, NB % _NBUF)
        gather_wait(NB, NB % _NBUF)
        scatter(NB, NB % _NBUF)

    for i in range(_NBUF):
        scatter_wait(NB - _NBUF + i, (NB - _NBUF + i) % _NBUF)

    plsc.subcore_barrier()
    pltpu.sync_copy(acc_sh.at[pl.ds(s * RPT, RPT)],
                    out_hbm.at[c, pl.ds(s * RPT, RPT)])


_ROWS_BLK = 2000
_GRID = N_NODES // _ROWS_BLK
_PROWS = R_SH // 2 // _GRID
_NPAIR = N_NODES // 2


def _h_body(x_ref, w_ref, h_ref):
    h_ref[...] = jnp.dot(x_ref[...], w_ref[...],
                         preferred_element_type=jnp.float32)


_h_call = pl.pallas_call(
    _h_body,
    grid=(_GRID,),
    in_specs=[
        pl.BlockSpec((_ROWS_BLK, D_IN), lambda i: (i, 0)),
        pl.BlockSpec((D_IN, D_HID), lambda i: (0, 0)),
    ],
    out_specs=pl.BlockSpec((_ROWS_BLK, D_HID), lambda i: (i, 0)),
    out_shape=jax.ShapeDtypeStruct((N_NODES, D_HID), jnp.float32),
)


def _scale_body(h_ref, deg_ref, s_ref):
    g = lax.rsqrt(deg_ref[...].astype(jnp.float32))
    s_ref[...] = h_ref[...] * g


_scale_call = pl.pallas_call(
    _scale_body,
    grid=(_GRID,),
    in_specs=[
        pl.BlockSpec((_ROWS_BLK, D_HID), lambda i: (i, 0)),
        pl.BlockSpec((_ROWS_BLK, 1), lambda i: (i, 0)),
    ],
    out_specs=pl.BlockSpec((_ROWS_BLK, D_HID), lambda i: (i, 0)),
    out_shape=jax.ShapeDtypeStruct((N_NODES, D_HID), jnp.float32),
)


def _final_body(acc_ref, deg_ref, b_ref, wf_ref, bf_ref, out_ref):
    i = pl.program_id(0)
    g = lax.rsqrt(deg_ref[...].astype(jnp.float32))
    agg = g * (acc_ref[0] + acc_ref[1])
    o = jnp.maximum(agg + b_ref[...], 0.0)
    p = lax.broadcasted_iota(jnp.int32, (_PROWS, 2 * D_HID), 0) + i * _PROWS
    o = jnp.where(p < _NPAIR, o, 0.0)
    o_r = o.astype(jnp.bfloat16).astype(jnp.float32)
    wf_r = wf_ref[...].astype(jnp.bfloat16).astype(jnp.float32)
    col = jnp.sum(o_r, axis=0, keepdims=True)
    part = jnp.sum(col * wf_r, axis=1, keepdims=True)

    @pl.when(i == 0)
    def _init():
        out_ref[...] = jnp.zeros_like(out_ref)

    out_ref[...] += part

    @pl.when(i == _GRID - 1)
    def _finish():
        out_ref[...] = out_ref[...] / float(N_NODES) + bf_ref[...]


_final_call = pl.pallas_call(
    _final_body,
    grid=(_GRID,),
    in_specs=[
        pl.BlockSpec((NC, _PROWS, 2 * D_HID), lambda i: (0, i, 0)),
        pl.BlockSpec((_PROWS, 2 * D_HID), lambda i: (i, 0)),
        pl.BlockSpec((1, 2 * D_HID), lambda i: (0, 0)),
        pl.BlockSpec((1, 2 * D_HID), lambda i: (0, 0)),
        pl.BlockSpec((1, 1), lambda i: (0, 0)),
    ],
    out_specs=pl.BlockSpec((1, 1), lambda i: (0, 0)),
    out_shape=jax.ShapeDtypeStruct((1, 1), jnp.float32),
)


@functools.cache
def _sc_kernels():
    mesh = plsc.VectorSubcoreMesh(core_axis_name="c", subcore_axis_name="s",
                                  num_cores=NC, num_subcores=NS)
    params = pltpu.CompilerParams(use_tc_tiling_on_sc=False)
    deg_kernel = pl.kernel(
        _deg_body,
        out_type=jax.ShapeDtypeStruct((NC, R_SH), jnp.float32),
        mesh=mesh,
        compiler_params=params,
        scratch_types=[
            pltpu.VMEM((NB + 1, BLK), jnp.int32),
            pltpu.VMEM((BLK,), jnp.float32),
            pltpu.VMEM((RPT,), jnp.float32),
            pltpu.VMEM_SHARED((R_SH,), jnp.float32),
            pltpu.SemaphoreType.DMA,
        ],
    )
    msg_kernel = pl.kernel(
        _msg_body,
        out_type=jax.ShapeDtypeStruct((NC, R_SH, D_HID), jnp.float32),
        mesh=mesh,
        compiler_params=params,
        scratch_types=(
            [pltpu.VMEM((NB + 1, BLK), jnp.int32)] * 2
            + [pltpu.VMEM((BLK, D_HID), jnp.float32)] * (_NBUF + 1)
            + [pltpu.VMEM_SHARED((R_SH, D_HID), jnp.float32)]
            + [pltpu.SemaphoreType.DMA] * (2 * _NBUF)
        ),
    )
    return deg_kernel, msg_kernel


def kernel(x, edge_index, W, b, Wf, bf):
    deg_kernel, msg_kernel = _sc_kernels()
    ei = edge_index.astype(jnp.int32)
    sl_src = jnp.concatenate(
        [jnp.arange(N_NODES, dtype=jnp.int32),
         jnp.zeros((SPAD,), jnp.int32)]).reshape(NBS, BLK)
    sl_dst = jnp.concatenate(
        [jnp.arange(N_NODES, dtype=jnp.int32),
         jnp.full((SPAD,), N_NODES, jnp.int32)]).reshape(NBS, BLK)
    ei3 = jnp.concatenate(
        [ei.reshape(2, N_EDGES // BLK, BLK),
         jnp.stack([sl_src, sl_dst])], axis=1)
    h_arr = _h_call(x, W)
    degp = deg_kernel(ei3)
    degt = degp[0] + degp[1]
    degc = degt[:N_NODES, None].astype(jnp.bfloat16)
    degc128 = jnp.broadcast_to(
        degt.reshape(R_SH // 2, 2, 1),
        (R_SH // 2, 2, D_HID)).reshape(R_SH // 2, 2 * D_HID).astype(jnp.bfloat16)

    s_arr = _scale_call(h_arr, degc)
    accp = msg_kernel(ei3, s_arr)
    accp2 = accp.reshape(NC, R_SH // 2, 2 * D_HID)

    b128 = jnp.concatenate([b, b]).reshape(1, 2 * D_HID)
    wf128 = jnp.concatenate([Wf[:, 0], Wf[:, 0]]).reshape(1, 2 * D_HID)
    res = _final_call(accp2, degc128, b128, wf128, bf.reshape(1, 1))
    return res.reshape(1)
